# baseline (device time: 213988 ns/iter reference)
import jax
import jax.numpy as jnp
from jax import lax
from jax.experimental import pallas as pl
from jax.experimental.pallas import tpu as pltpu

B, H, D, BS = 32, 16, 128, 32
PAGES_LOCAL = 256
NB = 256
SCALE = D ** -0.5
NEG = -1e30
CH = 32
NBUF = 2
MAXC = B * 2


def _partials_body(q_ref, pages_ref, starts_ref, nums_ref,
                   crow_ref, coff_ref, total_ref, k_hbm, v_hbm,
                   acc_ref, m_ref, l_ref,
                   k_buf, v_buf, sems):
    T = total_ref[0]

    def issue_chunk(tc, buf):
        row = crow_ref[tc]
        off = coff_ref[tc] * CH
        st = starts_ref[row]
        n = nums_ref[row]
        for u in range(CH):
            @pl.when(off + u < n)
            def _(u=u):
                idx = pages_ref[row, jnp.minimum(st + off + u, NB - 1)]
                pltpu.make_async_copy(
                    k_hbm.at[idx], k_buf.at[buf].at[pl.ds(u * BS, BS)],
                    sems.at[0, buf]).start()
                pltpu.make_async_copy(
                    v_hbm.at[idx], v_buf.at[buf].at[pl.ds(u * BS, BS)],
                    sems.at[1, buf]).start()

    def wait_chunk(tc, buf):
        off = coff_ref[tc] * CH
        n = nums_ref[crow_ref[tc]]
        for u in range(CH):
            @pl.when(off + u < n)
            def _(u=u):
                pltpu.make_async_copy(
                    k_hbm.at[0], k_buf.at[buf].at[pl.ds(u * BS, BS)],
                    sems.at[0, buf]).wait()
                pltpu.make_async_copy(
                    k_hbm.at[0], v_buf.at[buf].at[pl.ds(u * BS, BS)],
                    sems.at[1, buf]).wait()

    m_ref[...] = jnp.full((B, H), NEG, jnp.float32)
    l_ref[...] = jnp.zeros((B, H), jnp.float32)
    acc_ref[...] = jnp.zeros((B, H, D), jnp.float32)

    for w in range(NBUF - 1):
        @pl.when(w < T)
        def _(w=w):
            issue_chunk(w, w)

    def chunk_step(t, _):
        buf = lax.rem(t, NBUF)
        row = crow_ref[t]
        off = coff_ref[t] * CH
        n = nums_ref[row]

        @pl.when(t + NBUF - 1 < T)
        def _():
            issue_chunk(t + NBUF - 1, lax.rem(t + NBUF - 1, NBUF))

        wait_chunk(t, buf)
        q = q_ref[row, 0]
        ks = k_buf[buf]
        s = jnp.sum(q[None, :, :] * ks, axis=-1) * SCALE
        rows = lax.broadcasted_iota(jnp.int32, (CH * BS, H), 0)
        valid = (off + rows // BS) < n
        s = jnp.where(valid, s, NEG)
        m = m_ref[pl.ds(row, 1), :]
        l = l_ref[pl.ds(row, 1), :]
        acc = acc_ref[row]
        m_new = jnp.maximum(m, jnp.max(s, axis=0, keepdims=True))
        alpha = jnp.exp(m - m_new)
        pexp = jnp.where(valid, jnp.exp(s - m_new), 0.0)
        vs = v_buf[buf]
        pv = jnp.sum(pexp[:, :, None] * vs, axis=0)
        m_ref[pl.ds(row, 1), :] = m_new
        l_ref[pl.ds(row, 1), :] = alpha * l + jnp.sum(pexp, axis=0,
                                                      keepdims=True)
        acc_ref[row] = acc * jnp.reshape(alpha, (H, 1)) + pv
        return 0

    lax.fori_loop(0, T, chunk_step, 0)


def _combine_body(acc_ref, m_ref, l_ref, out_ref,
                  sacc, sm, sl, racc, rm, rl, send_sems, recv_sems):
    x = lax.axis_index("x")
    y = lax.axis_index("y")
    z = lax.axis_index("z")
    nbrs = ((x, y, 1 - z), (x, 1 - y, z), (1 - x, y, z))

    bsem = pltpu.get_barrier_semaphore()
    for nbr in nbrs:
        pl.semaphore_signal(bsem, inc=1, device_id=nbr,
                            device_id_type=pl.DeviceIdType.MESH)
    pl.semaphore_wait(bsem, 3)

    cur_acc = acc_ref[...]
    cur_m = m_ref[...]
    cur_l = l_ref[...]
    for r, nbr in enumerate(nbrs):
        if r == 0:
            srcs = (acc_ref, m_ref, l_ref)
        else:
            sacc[...] = cur_acc
            sm[...] = cur_m
            sl[...] = cur_l
            srcs = (sacc, sm, sl)
        copies = []
        for j, (src, dst) in enumerate(
            zip(srcs, (racc.at[r], rm.at[r], rl.at[r]))
        ):
            cp = pltpu.make_async_remote_copy(
                src_ref=src, dst_ref=dst,
                send_sem=send_sems.at[r, j], recv_sem=recv_sems.at[r, j],
                device_id=nbr, device_id_type=pl.DeviceIdType.MESH,
            )
            cp.start()
            copies.append(cp)
        for cp in copies:
            cp.wait()
        o_m = rm[r]
        o_l = rl[r]
        o_acc = racc[r]
        mx = jnp.maximum(cur_m, o_m)
        wa = jnp.exp(cur_m - mx)
        wb = jnp.exp(o_m - mx)
        cur_acc = cur_acc * wa[:, :, None] + o_acc * wb[:, :, None]
        cur_l = cur_l * wa + o_l * wb
        cur_m = mx

    out_ref[:, 0, :, :] = cur_acc / cur_l[:, :, None]


def kernel(Q, K, V, bt, lens):
    x = lax.axis_index("x")
    y = lax.axis_index("y")
    quarter = 2 * x + y

    base = lax.axis_index("z") * PAGES_LOCAL
    col = jnp.arange(NB, dtype=jnp.int32)[None, :]
    loc = bt - base
    owned = (col < lens[:, None]) & (loc >= 0) & (loc < PAGES_LOCAL)
    key = (
        jnp.where(owned, 0, 1 << 20)
        + (col << 10)
        + jnp.clip(loc, 0, PAGES_LOCAL - 1)
    )
    pages = (jnp.sort(key, axis=1) & 1023).astype(jnp.int32)
    counts = jnp.sum(owned, axis=1).astype(jnp.int32)

    starts = (counts * quarter) // 4
    nums = (counts * (quarter + 1)) // 4 - starts

    nc = (nums + CH - 1) // CH
    P = jnp.concatenate([jnp.zeros((1,), jnp.int32), jnp.cumsum(nc)]
                        ).astype(jnp.int32)
    total = P[B:B + 1]
    tc = jnp.arange(MAXC, dtype=jnp.int32)[:, None]
    le = P[None, :B] <= tc
    crow = (jnp.sum(le, axis=1) - 1).astype(jnp.int32)
    rowstart = jnp.max(jnp.where(le, P[None, :B], 0), axis=1)
    coff = (tc[:, 0] - rowstart).astype(jnp.int32)

    acc, m, l = pl.pallas_call(
        _partials_body,
        in_specs=[
            pl.BlockSpec(memory_space=pltpu.VMEM),
            pl.BlockSpec(memory_space=pltpu.SMEM),
            pl.BlockSpec(memory_space=pltpu.SMEM),
            pl.BlockSpec(memory_space=pltpu.SMEM),
            pl.BlockSpec(memory_space=pltpu.SMEM),
            pl.BlockSpec(memory_space=pltpu.SMEM),
            pl.BlockSpec(memory_space=pltpu.SMEM),
            pl.BlockSpec(memory_space=pl.ANY),
            pl.BlockSpec(memory_space=pl.ANY),
        ],
        out_specs=[
            pl.BlockSpec(memory_space=pltpu.VMEM),
            pl.BlockSpec(memory_space=pltpu.VMEM),
            pl.BlockSpec(memory_space=pltpu.VMEM),
        ],
        out_shape=[
            jax.ShapeDtypeStruct((B, H, D), jnp.float32),
            jax.ShapeDtypeStruct((B, H), jnp.float32),
            jax.ShapeDtypeStruct((B, H), jnp.float32),
        ],
        scratch_shapes=[
            pltpu.VMEM((NBUF, CH * BS, H, D), jnp.float32),
            pltpu.VMEM((NBUF, CH * BS, H, D), jnp.float32),
            pltpu.SemaphoreType.DMA((2, NBUF)),
        ],
        compiler_params=pltpu.CompilerParams(
            vmem_limit_bytes=100 * 1024 * 1024,
        ),
    )(Q, pages, starts, nums, crow, coff, total, K, V)

    return pl.pallas_call(
        _combine_body,
        in_specs=[pl.BlockSpec(memory_space=pltpu.VMEM)] * 3,
        out_specs=pl.BlockSpec(memory_space=pltpu.VMEM),
        out_shape=jax.ShapeDtypeStruct((B, 1, H, D), jnp.float32),
        scratch_shapes=[
            pltpu.VMEM((B, H, D), jnp.float32),
            pltpu.VMEM((B, H), jnp.float32),
            pltpu.VMEM((B, H), jnp.float32),
            pltpu.VMEM((3, B, H, D), jnp.float32),
            pltpu.VMEM((3, B, H), jnp.float32),
            pltpu.VMEM((3, B, H), jnp.float32),
            pltpu.SemaphoreType.DMA((3, 3)),
            pltpu.SemaphoreType.DMA((3, 3)),
        ],
        compiler_params=pltpu.CompilerParams(collective_id=0),
    )(acc, m, l)


# device time: 179013 ns/iter; 1.1954x vs baseline; 1.1954x over previous
import jax
import jax.numpy as jnp
from jax import lax
from jax.experimental import pallas as pl
from jax.experimental.pallas import tpu as pltpu

B, H, D, BS = 32, 16, 128, 32
PAGES_LOCAL = 256
NB = 256
SCALE = D ** -0.5
NEG = -1e30
CH = 16
NBUF = 3
MAXC = B * 4


def _partials_body(q_ref, pages_ref, starts_ref, nums_ref,
                   crow_ref, coff_ref, total_ref, k_hbm, v_hbm,
                   acc_ref, m_ref, l_ref,
                   k_buf, v_buf, sems):
    T = total_ref[0]

    def issue_chunk(tc, buf):
        row = crow_ref[tc]
        off = coff_ref[tc] * CH
        st = starts_ref[row]
        n = nums_ref[row]
        for u in range(CH):
            @pl.when(off + u < n)
            def _(u=u):
                idx = pages_ref[row, jnp.minimum(st + off + u, NB - 1)]
                pltpu.make_async_copy(
                    k_hbm.at[idx], k_buf.at[buf].at[pl.ds(u * BS, BS)],
                    sems.at[0, buf]).start()
                pltpu.make_async_copy(
                    v_hbm.at[idx], v_buf.at[buf].at[pl.ds(u * BS, BS)],
                    sems.at[1, buf]).start()

    def wait_chunk(tc, buf):
        off = coff_ref[tc] * CH
        n = nums_ref[crow_ref[tc]]
        for u in range(CH):
            @pl.when(off + u < n)
            def _(u=u):
                pltpu.make_async_copy(
                    k_hbm.at[0], k_buf.at[buf].at[pl.ds(u * BS, BS)],
                    sems.at[0, buf]).wait()
                pltpu.make_async_copy(
                    k_hbm.at[0], v_buf.at[buf].at[pl.ds(u * BS, BS)],
                    sems.at[1, buf]).wait()

    m_ref[...] = jnp.full((B, H), NEG, jnp.float32)
    l_ref[...] = jnp.zeros((B, H), jnp.float32)
    acc_ref[...] = jnp.zeros((B, H, D), jnp.float32)

    for w in range(NBUF - 1):
        @pl.when(w < T)
        def _(w=w):
            issue_chunk(w, w)

    def chunk_step(t, _):
        buf = lax.rem(t, NBUF)
        row = crow_ref[t]
        off = coff_ref[t] * CH
        n = nums_ref[row]

        @pl.when(t + NBUF - 1 < T)
        def _():
            issue_chunk(t + NBUF - 1, lax.rem(t + NBUF - 1, NBUF))

        wait_chunk(t, buf)
        q = q_ref[row, 0]
        ks = k_buf[buf]
        s = jnp.sum(q[None, :, :] * ks, axis=-1) * SCALE
        rows = lax.broadcasted_iota(jnp.int32, (CH * BS, H), 0)
        valid = (off + rows // BS) < n
        s = jnp.where(valid, s, NEG)
        m = m_ref[pl.ds(row, 1), :]
        l = l_ref[pl.ds(row, 1), :]
        acc = acc_ref[row]
        m_new = jnp.maximum(m, jnp.max(s, axis=0, keepdims=True))
        alpha = jnp.exp(m - m_new)
        pexp = jnp.where(valid, jnp.exp(s - m_new), 0.0)
        vs = v_buf[buf]
        pv = jnp.sum(pexp[:, :, None] * vs, axis=0)
        m_ref[pl.ds(row, 1), :] = m_new
        l_ref[pl.ds(row, 1), :] = alpha * l + jnp.sum(pexp, axis=0,
                                                      keepdims=True)
        acc_ref[row] = acc * jnp.reshape(alpha, (H, 1)) + pv
        return 0

    lax.fori_loop(0, T, chunk_step, 0)


def _combine_body(acc_ref, m_ref, l_ref, out_ref,
                  sacc, sm, sl, racc, rm, rl, send_sems, recv_sems):
    x = lax.axis_index("x")
    y = lax.axis_index("y")
    z = lax.axis_index("z")
    nbrs = ((x, y, 1 - z), (x, 1 - y, z), (1 - x, y, z))

    bsem = pltpu.get_barrier_semaphore()
    for nbr in nbrs:
        pl.semaphore_signal(bsem, inc=1, device_id=nbr,
                            device_id_type=pl.DeviceIdType.MESH)
    pl.semaphore_wait(bsem, 3)

    cur_acc = acc_ref[...]
    cur_m = m_ref[...]
    cur_l = l_ref[...]
    for r, nbr in enumerate(nbrs):
        if r == 0:
            srcs = (acc_ref, m_ref, l_ref)
        else:
            sacc[...] = cur_acc
            sm[...] = cur_m
            sl[...] = cur_l
            srcs = (sacc, sm, sl)
        copies = []
        for j, (src, dst) in enumerate(
            zip(srcs, (racc.at[r], rm.at[r], rl.at[r]))
        ):
            cp = pltpu.make_async_remote_copy(
                src_ref=src, dst_ref=dst,
                send_sem=send_sems.at[r, j], recv_sem=recv_sems.at[r, j],
                device_id=nbr, device_id_type=pl.DeviceIdType.MESH,
            )
            cp.start()
            copies.append(cp)
        for cp in copies:
            cp.wait()
        o_m = rm[r]
        o_l = rl[r]
        o_acc = racc[r]
        mx = jnp.maximum(cur_m, o_m)
        wa = jnp.exp(cur_m - mx)
        wb = jnp.exp(o_m - mx)
        cur_acc = cur_acc * wa[:, :, None] + o_acc * wb[:, :, None]
        cur_l = cur_l * wa + o_l * wb
        cur_m = mx

    out_ref[:, 0, :, :] = cur_acc / cur_l[:, :, None]


def kernel(Q, K, V, bt, lens):
    x = lax.axis_index("x")
    y = lax.axis_index("y")
    quarter = 2 * x + y

    base = lax.axis_index("z") * PAGES_LOCAL
    col = jnp.arange(NB, dtype=jnp.int32)[None, :]
    loc = bt - base
    owned = (col < lens[:, None]) & (loc >= 0) & (loc < PAGES_LOCAL)
    key = (
        jnp.where(owned, 0, 1 << 20)
        + (col << 10)
        + jnp.clip(loc, 0, PAGES_LOCAL - 1)
    )
    pages = (jnp.sort(key, axis=1) & 1023).astype(jnp.int32)
    counts = jnp.sum(owned, axis=1).astype(jnp.int32)

    starts = (counts * quarter) // 4
    nums = (counts * (quarter + 1)) // 4 - starts

    nc = (nums + CH - 1) // CH
    P = jnp.concatenate([jnp.zeros((1,), jnp.int32), jnp.cumsum(nc)]
                        ).astype(jnp.int32)
    total = P[B:B + 1]
    tc = jnp.arange(MAXC, dtype=jnp.int32)[:, None]
    le = P[None, :B] <= tc
    crow = (jnp.sum(le, axis=1) - 1).astype(jnp.int32)
    rowstart = jnp.max(jnp.where(le, P[None, :B], 0), axis=1)
    coff = (tc[:, 0] - rowstart).astype(jnp.int32)

    acc, m, l = pl.pallas_call(
        _partials_body,
        in_specs=[
            pl.BlockSpec(memory_space=pltpu.VMEM),
            pl.BlockSpec(memory_space=pltpu.SMEM),
            pl.BlockSpec(memory_space=pltpu.SMEM),
            pl.BlockSpec(memory_space=pltpu.SMEM),
            pl.BlockSpec(memory_space=pltpu.SMEM),
            pl.BlockSpec(memory_space=pltpu.SMEM),
            pl.BlockSpec(memory_space=pltpu.SMEM),
            pl.BlockSpec(memory_space=pl.ANY),
            pl.BlockSpec(memory_space=pl.ANY),
        ],
        out_specs=[
            pl.BlockSpec(memory_space=pltpu.VMEM),
            pl.BlockSpec(memory_space=pltpu.VMEM),
            pl.BlockSpec(memory_space=pltpu.VMEM),
        ],
        out_shape=[
            jax.ShapeDtypeStruct((B, H, D), jnp.float32),
            jax.ShapeDtypeStruct((B, H), jnp.float32),
            jax.ShapeDtypeStruct((B, H), jnp.float32),
        ],
        scratch_shapes=[
            pltpu.VMEM((NBUF, CH * BS, H, D), jnp.float32),
            pltpu.VMEM((NBUF, CH * BS, H, D), jnp.float32),
            pltpu.SemaphoreType.DMA((2, NBUF)),
        ],
        compiler_params=pltpu.CompilerParams(
            vmem_limit_bytes=100 * 1024 * 1024,
        ),
    )(Q, pages, starts, nums, crow, coff, total, K, V)

    return pl.pallas_call(
        _combine_body,
        in_specs=[pl.BlockSpec(memory_space=pltpu.VMEM)] * 3,
        out_specs=pl.BlockSpec(memory_space=pltpu.VMEM),
        out_shape=jax.ShapeDtypeStruct((B, 1, H, D), jnp.float32),
        scratch_shapes=[
            pltpu.VMEM((B, H, D), jnp.float32),
            pltpu.VMEM((B, H), jnp.float32),
            pltpu.VMEM((B, H), jnp.float32),
            pltpu.VMEM((3, B, H, D), jnp.float32),
            pltpu.VMEM((3, B, H), jnp.float32),
            pltpu.VMEM((3, B, H), jnp.float32),
            pltpu.SemaphoreType.DMA((3, 3)),
            pltpu.SemaphoreType.DMA((3, 3)),
        ],
        compiler_params=pltpu.CompilerParams(collective_id=0),
    )(acc, m, l)
